# baseline (speedup 1.0000x reference)
import jax
import jax.numpy as jnp
from jax.experimental import pallas as pl
from jax.experimental.pallas import tpu as pltpu

N_R = 4096
N_RP1 = 4096
D_OUT = 32
BM = 256


def _elu(v):
    return jnp.where(v > 0, v, jnp.exp(jnp.minimum(v, 0.0)) - 1.0)


def _fused_kernel(n11_ref, n21_ref, x1_ref, x2_ref, w1_ref, w2_ref,
                  out_ref, xw1_ref, xw2_ref):
    i = pl.program_id(0)

    @pl.when(i == 0)
    def _project():
        xw1_ref[...] = jnp.dot(x1_ref[...], w1_ref[...],
                               preferred_element_type=jnp.float32)
        xw2_ref[...] = jnp.dot(x2_ref[...], w2_ref[...],
                               preferred_element_type=jnp.float32)

    up = jnp.dot(n11_ref[...], xw1_ref[...],
                 preferred_element_type=jnp.float32)
    cob = jnp.dot(n21_ref[...], xw2_ref[...],
                  preferred_element_type=jnp.float32)
    out_ref[...] = _elu(up) + _elu(cob)


def kernel(x_1, x_2, neighborhood_1_to_1, neighborhood_2_to_1, W1, W2):
    grid = (N_R // BM,)
    return pl.pallas_call(
        _fused_kernel,
        grid=grid,
        in_specs=[
            pl.BlockSpec((BM, N_R), lambda i: (i, 0)),
            pl.BlockSpec((BM, N_RP1), lambda i: (i, 0)),
            pl.BlockSpec((N_R, x_1.shape[1]), lambda i: (0, 0)),
            pl.BlockSpec((N_RP1, x_2.shape[1]), lambda i: (0, 0)),
            pl.BlockSpec((x_1.shape[1], D_OUT), lambda i: (0, 0)),
            pl.BlockSpec((x_2.shape[1], D_OUT), lambda i: (0, 0)),
        ],
        out_specs=pl.BlockSpec((BM, D_OUT), lambda i: (i, 0)),
        out_shape=jax.ShapeDtypeStruct((N_R, D_OUT), jnp.float32),
        scratch_shapes=[
            pltpu.VMEM((N_R, D_OUT), jnp.float32),
            pltpu.VMEM((N_RP1, D_OUT), jnp.float32),
        ],
        compiler_params=pltpu.CompilerParams(
            dimension_semantics=("arbitrary",),
        ),
    )(neighborhood_1_to_1, neighborhood_2_to_1, x_1, x_2, W1, W2)
